# baseline (speedup 1.0000x reference)
import jax
import jax.numpy as jnp
from jax import lax
from jax.experimental import pallas as pl
from jax.experimental.pallas import tpu as pltpu
from jax.experimental.pallas import tpu_sc as plsc

_VOCAB = 1000000
_EMBED = 64
_B, _L = 1024, 200
_NW = 32
_ROWS = _B * _L
_RPW = _ROWS // _NW
_BPW = _B // _NW
_LP = 208
_NGRP = _LP // 16

_C = 2048
_HALF = _C // 2
_NBLK = (_VOCAB + _C - 1) // _C
_WROWS = _NBLK * _HALF


def _embed_kernel(x_hbm, tokw_hbm, pos_hbm, out_hbm,
                  iw_all, par_all, wide0, wide1, stage0, stage1,
                  pos_v, pos_t,
                  gsem0, gsem1, wsem0, wsem1):
    c = lax.axis_index("c")
    s = lax.axis_index("s")
    wid = s * 2 + c
    wides = (wide0, wide1)
    stages = (stage0, stage1)
    gsems = (gsem0, gsem1)
    wsems = (wsem0, wsem1)
    iota = lax.iota(jnp.int32, 16)

    pltpu.sync_copy(pos_hbm.at[pl.ds(0, _L)], pos_v.at[pl.ds(0, _L)])
    pltpu.sync_copy(x_hbm.at[pl.ds(wid * _RPW, _RPW)], iw_all.at[pl.ds(0, _RPW)])

    def xform(k, carry):
        sl = pl.ds(16 * k, 16)
        iv = iw_all[sl]
        rem = iv & (_C - 1)
        par_all[sl] = (rem >> 10) << 6
        iw_all[sl] = ((iv >> 11) << 10) + (rem & (_HALF - 1))
        return carry

    lax.fori_loop(0, _RPW // 16, xform, 0)
    par_all[pl.ds(_RPW, 16)] = jnp.zeros((16,), jnp.int32)
    iw_all[pl.ds(_RPW, 16)] = jnp.zeros((16,), jnp.int32)

    def pos_tr(d, carry):
        def pos_tr_g(g, carry2):
            v = plsc.load_gather(pos_v, [iota + 16 * g, jnp.full((16,), d, jnp.int32)])
            pos_t[d, pl.ds(16 * g, 16)] = v
            return carry2
        lax.fori_loop(0, _NGRP, pos_tr_g, 0)
        return carry

    lax.fori_loop(0, _EMBED, pos_tr, 0)

    descs = {}

    def start(sc):
        b = sc & 1
        if sc >= 2:
            pltpu.make_async_copy(out_hbm.at[0], stages[b].at[pl.ds(0, _L)],
                                  wsems[b]).wait()
        off = sc * _L
        descs[sc] = [
            pltpu.async_copy(
                tokw_hbm.at[iw_all.at[pl.ds(off, 128)]],
                wides[b].at[pl.ds(0, 128)], gsems[b]),
            pltpu.async_copy(
                tokw_hbm.at[iw_all.at[pl.ds(off + 128, 72)]],
                wides[b].at[pl.ds(128, 72)], gsems[b]),
        ]

    def process(sc):
        b = sc & 1
        for d in descs[sc]:
            d.wait()
        wide = wides[b]
        stage = stages[b]
        off = sc * _L

        def grp(g, carry):
            row_idx = iota + 16 * g
            par = par_all[pl.ds(off + 16 * g, 16)]

            def dloop(d, carry2):
                tok = plsc.load_gather(wide, [row_idx, par + d])
                p = pos_t[d, pl.ds(16 * g, 16)]
                plsc.store_scatter(stage, [row_idx, jnp.full((16,), d, jnp.int32)],
                                   tok + p)
                return carry2

            lax.fori_loop(0, _EMBED, dloop, 0, unroll=4)
            return carry

        lax.fori_loop(0, _NGRP, grp, 0)
        pltpu.async_copy(stage.at[pl.ds(0, _L)],
                         out_hbm.at[wid * _BPW + sc], wsems[b])

    start(0)
    for sc in range(_BPW):
        if sc + 1 < _BPW:
            start(sc + 1)
        process(sc)
    pltpu.make_async_copy(out_hbm.at[0], stage0.at[pl.ds(0, _L)], wsem0).wait()
    pltpu.make_async_copy(out_hbm.at[0], stage1.at[pl.ds(0, _L)], wsem1).wait()


def _repack_kernel(src_ref, out_ref):
    t = jnp.transpose(src_ref[...])
    out_ref[:, 0:_EMBED] = t[0:_HALF, :]
    out_ref[:, _EMBED:2 * _EMBED] = t[_HALF:, :]


def _repack(tok_t):
    return pl.pallas_call(
        _repack_kernel,
        grid=(_NBLK,),
        in_specs=[pl.BlockSpec((_EMBED, _C), lambda g: (0, g))],
        out_specs=pl.BlockSpec((_HALF, 2 * _EMBED), lambda g: (g, 0)),
        out_shape=jax.ShapeDtypeStruct((_WROWS, 2 * _EMBED), jnp.float32),
    )(tok_t)


def _embed(xf, tokw, pos_table):
    mesh = plsc.VectorSubcoreMesh(core_axis_name="c", subcore_axis_name="s")
    return pl.kernel(
        _embed_kernel,
        out_type=jax.ShapeDtypeStruct((_B, _L, _EMBED), jnp.float32),
        mesh=mesh,
        scratch_types=[
            pltpu.VMEM((_RPW + 16,), jnp.int32),
            pltpu.VMEM((_RPW + 16,), jnp.int32),
            pltpu.VMEM((_LP, 128), jnp.float32),
            pltpu.VMEM((_LP, 128), jnp.float32),
            pltpu.VMEM((_LP, _EMBED), jnp.float32),
            pltpu.VMEM((_LP, _EMBED), jnp.float32),
            pltpu.VMEM((_LP, _EMBED), jnp.float32),
            pltpu.VMEM((_EMBED, _LP), jnp.float32),
            pltpu.SemaphoreType.DMA,
            pltpu.SemaphoreType.DMA,
            pltpu.SemaphoreType.DMA,
            pltpu.SemaphoreType.DMA,
        ],
        compiler_params=pltpu.CompilerParams(use_tc_tiling_on_sc=False,
                                             needs_layout_passes=False),
    )(xf, tokw, pos_table)


@jax.jit
def _run(x, tok_table, pos_table):
    xf = jnp.reshape(x, (_ROWS,)).astype(jnp.int32)
    tokw = _repack(jnp.transpose(tok_table))
    return _embed(xf, tokw, pos_table)


def kernel(x, tok_table, pos_table):
    return _run(x, tok_table, pos_table)


# speedup vs baseline: 2.2018x; 2.2018x over previous
import jax
import jax.numpy as jnp
from jax import lax
from jax.experimental import pallas as pl
from jax.experimental.pallas import tpu as pltpu
from jax.experimental.pallas import tpu_sc as plsc

_VOCAB = 1000000
_EMBED = 64
_B, _L = 1024, 200
_NW = 32
_ROWS = _B * _L
_RPW = _ROWS // _NW
_BPW = _B // _NW

_C = 4096
_HALF = _C // 2
_NBLK = (_VOCAB + _C - 1) // _C
_TROWS = _NBLK * _C


def _repack_kernel(src_ref, out_ref):
    t = jnp.transpose(src_ref[...])
    out_ref[:, 0:_EMBED] = t[0:_HALF, :]
    out_ref[:, _EMBED:2 * _EMBED] = t[_HALF:, :]


def _repack(tok_t):
    return pl.pallas_call(
        _repack_kernel,
        grid=(_NBLK,),
        in_specs=[pl.BlockSpec((_EMBED, _C), lambda g: (0, g))],
        out_specs=pl.BlockSpec((_HALF, 2 * _EMBED), lambda g: (g, 0)),
        out_shape=jax.ShapeDtypeStruct((_TROWS // 2, 2 * _EMBED), jnp.float32),
    )(tok_t)


def _embed_kernel(x_hbm, tok_hbm, pos_hbm, out_hbm,
                  idx_all, buf0, buf1, stage0, stage1, pos_v,
                  gsem0, gsem1, wsem0, wsem1):
    c = lax.axis_index("c")
    s = lax.axis_index("s")
    wid = s * 2 + c
    bufs = (buf0, buf1)
    stages = (stage0, stage1)
    gsems = (gsem0, gsem1)
    wsems = (wsem0, wsem1)

    pltpu.sync_copy(pos_hbm.at[pl.ds(0, _L)], pos_v)
    pltpu.sync_copy(x_hbm.at[pl.ds(wid * _RPW, _RPW)], idx_all)

    def xform(k, carry):
        sl = pl.ds(16 * k, 16)
        iv = idx_all[sl]
        rem = iv & (_C - 1)
        idx_all[sl] = ((iv >> 12) << 12) + ((rem & (_HALF - 1)) << 1) + (rem >> 11)
        return carry

    lax.fori_loop(0, _RPW // 16, xform, 0)

    descs = {}

    def start(sc):
        b = sc & 1
        if sc >= 2:
            pltpu.make_async_copy(out_hbm.at[pl.ds(0, _L // 2)],
                                  stages[b], wsems[b]).wait()
        off = sc * _L
        descs[sc] = [
            pltpu.async_copy(
                tok_hbm.at[idx_all.at[pl.ds(off, 128)]],
                bufs[b].at[pl.ds(0, 128)], gsems[b]),
            pltpu.async_copy(
                tok_hbm.at[idx_all.at[pl.ds(off + 128, 72)]],
                bufs[b].at[pl.ds(128, 72)], gsems[b]),
        ]

    def process(sc):
        b = sc & 1
        for d in descs[sc]:
            d.wait()
        buf = bufs[b]
        stage = stages[b]

        def row_body(p, carry):
            for half in range(2):
                for jj in range(_EMBED // 16):
                    dsl = pl.ds(half * _EMBED + 16 * jj, 16)
                    stage[p, dsl] = (buf[2 * p + half, pl.ds(16 * jj, 16)]
                                     + pos_v[2 * p + half, pl.ds(16 * jj, 16)])
            return carry

        lax.fori_loop(0, _L // 2, row_body, 0)
        pltpu.async_copy(stage,
                         out_hbm.at[pl.ds((wid * _BPW + sc) * (_L // 2), _L // 2)],
                         wsems[b])

    start(0)
    for sc in range(_BPW):
        if sc + 1 < _BPW:
            start(sc + 1)
        process(sc)
    pltpu.make_async_copy(out_hbm.at[pl.ds(0, _L // 2)], stage0, wsem0).wait()
    pltpu.make_async_copy(out_hbm.at[pl.ds(0, _L // 2)], stage1, wsem1).wait()


def _embed(xf, tokc, pos_table):
    mesh = plsc.VectorSubcoreMesh(core_axis_name="c", subcore_axis_name="s")
    return pl.kernel(
        _embed_kernel,
        out_type=jax.ShapeDtypeStruct((_ROWS // 2, 2 * _EMBED), jnp.float32),
        mesh=mesh,
        scratch_types=[
            pltpu.VMEM((_RPW,), jnp.int32),
            pltpu.VMEM((_L, _EMBED), jnp.float32),
            pltpu.VMEM((_L, _EMBED), jnp.float32),
            pltpu.VMEM((_L // 2, 2 * _EMBED), jnp.float32),
            pltpu.VMEM((_L // 2, 2 * _EMBED), jnp.float32),
            pltpu.VMEM((_L, _EMBED), jnp.float32),
            pltpu.SemaphoreType.DMA,
            pltpu.SemaphoreType.DMA,
            pltpu.SemaphoreType.DMA,
            pltpu.SemaphoreType.DMA,
        ],
        compiler_params=pltpu.CompilerParams(use_tc_tiling_on_sc=False),
    )(xf, tokc, pos_table)


@jax.jit
def _run(x, tok_table, pos_table):
    xf = jnp.reshape(x, (_ROWS,)).astype(jnp.int32)
    tokw = _repack(jnp.transpose(tok_table))
    tokc = jnp.reshape(tokw, (_TROWS, _EMBED))
    out = _embed(xf, tokc, pos_table)
    return jnp.reshape(out, (_B, _L, _EMBED))


def kernel(x, tok_table, pos_table):
    return _run(x, tok_table, pos_table)


# speedup vs baseline: 2.5219x; 1.1454x over previous
import jax
import jax.numpy as jnp
from jax import lax
from jax.experimental import pallas as pl
from jax.experimental.pallas import tpu as pltpu
from jax.experimental.pallas import tpu_sc as plsc

_VOCAB = 1000000
_EMBED = 64
_B, _L = 1024, 200
_NW = 32
_ROWS = _B * _L
_RPW = _ROWS // _NW
_BPW = _B // _NW

_C = 8192
_LOGC = _C.bit_length() - 1
_HALF = _C // 2
_NBLK = (_VOCAB + _C - 1) // _C
_TROWS = _NBLK * _C


def _repack_kernel(src_ref, out_ref):
    t = jnp.transpose(src_ref[...])
    out_ref[:, 0:_EMBED] = t[0:_HALF, :]
    out_ref[:, _EMBED:2 * _EMBED] = t[_HALF:, :]


def _repack(tok_t):
    return pl.pallas_call(
        _repack_kernel,
        grid=(_NBLK,),
        in_specs=[pl.BlockSpec((_EMBED, _C), lambda g: (0, g))],
        out_specs=pl.BlockSpec((_HALF, 2 * _EMBED), lambda g: (g, 0)),
        out_shape=jax.ShapeDtypeStruct((_TROWS // 2, 2 * _EMBED), jnp.float32),
    )(tok_t)


def _embed_kernel(x_hbm, tok_hbm, pos_hbm, out_hbm,
                  idx_all, buf0, buf1, stage0, stage1, pos_v,
                  gsem0, gsem1, wsem0, wsem1):
    c = lax.axis_index("c")
    s = lax.axis_index("s")
    wid = s * 2 + c
    bufs = (buf0, buf1)
    stages = (stage0, stage1)
    gsems = (gsem0, gsem1)
    wsems = (wsem0, wsem1)

    pltpu.sync_copy(pos_hbm.at[pl.ds(0, _L)], pos_v)
    pltpu.sync_copy(x_hbm.at[pl.ds(wid * _RPW, _RPW)], idx_all)

    def xform(k, carry):
        sl = pl.ds(16 * k, 16)
        iv = idx_all[sl]
        rem = iv & (_C - 1)
        idx_all[sl] = ((iv >> _LOGC) << _LOGC) + ((rem & (_HALF - 1)) << 1) + (rem >> (_LOGC - 1))
        return carry

    lax.fori_loop(0, _RPW // 16, xform, 0)

    descs = {}

    def start(sc):
        b = sc & 1
        if sc >= 2:
            pltpu.make_async_copy(out_hbm.at[pl.ds(0, _L // 2)],
                                  stages[b], wsems[b]).wait()
        off = sc * _L
        descs[sc] = [
            pltpu.async_copy(
                tok_hbm.at[idx_all.at[pl.ds(off, 128)]],
                bufs[b].at[pl.ds(0, 128)], gsems[b]),
            pltpu.async_copy(
                tok_hbm.at[idx_all.at[pl.ds(off + 128, 72)]],
                bufs[b].at[pl.ds(128, 72)], gsems[b]),
        ]

    def process(sc):
        b = sc & 1
        for d in descs[sc]:
            d.wait()
        buf = bufs[b]
        stage = stages[b]

        def row_body(p, carry):
            for half in range(2):
                for jj in range(_EMBED // 16):
                    dsl = pl.ds(half * _EMBED + 16 * jj, 16)
                    stage[p, dsl] = (buf[2 * p + half, pl.ds(16 * jj, 16)]
                                     + pos_v[2 * p + half, pl.ds(16 * jj, 16)])
            return carry

        lax.fori_loop(0, _L // 2, row_body, 0)
        pltpu.async_copy(stage,
                         out_hbm.at[pl.ds((wid * _BPW + sc) * (_L // 2), _L // 2)],
                         wsems[b])

    start(0)
    for sc in range(_BPW):
        if sc + 1 < _BPW:
            start(sc + 1)
        process(sc)
    pltpu.make_async_copy(out_hbm.at[pl.ds(0, _L // 2)], stage0, wsem0).wait()
    pltpu.make_async_copy(out_hbm.at[pl.ds(0, _L // 2)], stage1, wsem1).wait()


def _embed(xf, tokc, pos_table):
    mesh = plsc.VectorSubcoreMesh(core_axis_name="c", subcore_axis_name="s")
    return pl.kernel(
        _embed_kernel,
        out_type=jax.ShapeDtypeStruct((_ROWS // 2, 2 * _EMBED), jnp.float32),
        mesh=mesh,
        scratch_types=[
            pltpu.VMEM((_RPW,), jnp.int32),
            pltpu.VMEM((_L, _EMBED), jnp.float32),
            pltpu.VMEM((_L, _EMBED), jnp.float32),
            pltpu.VMEM((_L // 2, 2 * _EMBED), jnp.float32),
            pltpu.VMEM((_L // 2, 2 * _EMBED), jnp.float32),
            pltpu.VMEM((_L, _EMBED), jnp.float32),
            pltpu.SemaphoreType.DMA,
            pltpu.SemaphoreType.DMA,
            pltpu.SemaphoreType.DMA,
            pltpu.SemaphoreType.DMA,
        ],
        compiler_params=pltpu.CompilerParams(use_tc_tiling_on_sc=False),
    )(xf, tokc, pos_table)


@jax.jit
def _run(x, tok_table, pos_table):
    xf = jnp.reshape(x, (_ROWS,)).astype(jnp.int32)
    tokw = _repack(jnp.transpose(tok_table))
    tokc = jnp.reshape(tokw, (_TROWS, _EMBED))
    out = _embed(xf, tokc, pos_table)
    return jnp.reshape(out, (_B, _L, _EMBED))


def kernel(x, tok_table, pos_table):
    return _run(x, tok_table, pos_table)


# speedup vs baseline: 2.7149x; 1.0765x over previous
import jax
import jax.numpy as jnp
from jax import lax
from jax.experimental import pallas as pl
from jax.experimental.pallas import tpu as pltpu
from jax.experimental.pallas import tpu_sc as plsc

_VOCAB = 1000000
_EMBED = 64
_B, _L = 1024, 200
_NW = 32
_ROWS = _B * _L
_RPW = _ROWS // _NW
_BPW = _B // _NW

_C = 16384
_LOGC = _C.bit_length() - 1
_HALF = _C // 2
_NBLK = (_VOCAB + _C - 1) // _C
_TROWS = _NBLK * _C


def _repack_kernel(src_ref, out_ref):
    t = jnp.transpose(src_ref[...])
    out_ref[:, 0:_EMBED] = t[0:_HALF, :]
    out_ref[:, _EMBED:2 * _EMBED] = t[_HALF:, :]


def _repack(tok_t):
    return pl.pallas_call(
        _repack_kernel,
        grid=(_NBLK,),
        in_specs=[pl.BlockSpec((_EMBED, _C), lambda g: (0, g))],
        out_specs=pl.BlockSpec((_HALF, 2 * _EMBED), lambda g: (g, 0)),
        out_shape=jax.ShapeDtypeStruct((_TROWS // 2, 2 * _EMBED), jnp.float32),
    )(tok_t)


def _embed_kernel(x_hbm, tok_hbm, pos_hbm, out_hbm,
                  idx_all, buf0, buf1, stage0, stage1, pos_v,
                  gsem0, gsem1, wsem0, wsem1):
    c = lax.axis_index("c")
    s = lax.axis_index("s")
    wid = s * 2 + c
    bufs = (buf0, buf1)
    stages = (stage0, stage1)
    gsems = (gsem0, gsem1)
    wsems = (wsem0, wsem1)

    pltpu.sync_copy(pos_hbm.at[pl.ds(0, _L)], pos_v)
    pltpu.sync_copy(x_hbm.at[pl.ds(wid * _RPW, _RPW)], idx_all)

    def xform(k, carry):
        sl = pl.ds(16 * k, 16)
        iv = idx_all[sl]
        rem = iv & (_C - 1)
        idx_all[sl] = ((iv >> _LOGC) << _LOGC) + ((rem & (_HALF - 1)) << 1) + (rem >> (_LOGC - 1))
        return carry

    lax.fori_loop(0, _RPW // 16, xform, 0)

    descs = {}

    def start(sc):
        b = sc & 1
        if sc >= 2:
            pltpu.make_async_copy(out_hbm.at[pl.ds(0, _L // 2)],
                                  stages[b], wsems[b]).wait()
        off = sc * _L
        descs[sc] = [
            pltpu.async_copy(
                tok_hbm.at[idx_all.at[pl.ds(off, 128)]],
                bufs[b].at[pl.ds(0, 128)], gsems[b]),
            pltpu.async_copy(
                tok_hbm.at[idx_all.at[pl.ds(off + 128, 72)]],
                bufs[b].at[pl.ds(128, 72)], gsems[b]),
        ]

    def process(sc):
        b = sc & 1
        for d in descs[sc]:
            d.wait()
        buf = bufs[b]
        stage = stages[b]

        def row_body(p, carry):
            for half in range(2):
                for jj in range(_EMBED // 16):
                    dsl = pl.ds(half * _EMBED + 16 * jj, 16)
                    stage[p, dsl] = (buf[2 * p + half, pl.ds(16 * jj, 16)]
                                     + pos_v[2 * p + half, pl.ds(16 * jj, 16)])
            return carry

        lax.fori_loop(0, _L // 2, row_body, 0)
        pltpu.async_copy(stage,
                         out_hbm.at[pl.ds((wid * _BPW + sc) * (_L // 2), _L // 2)],
                         wsems[b])

    start(0)
    for sc in range(_BPW):
        if sc + 1 < _BPW:
            start(sc + 1)
        process(sc)
    pltpu.make_async_copy(out_hbm.at[pl.ds(0, _L // 2)], stage0, wsem0).wait()
    pltpu.make_async_copy(out_hbm.at[pl.ds(0, _L // 2)], stage1, wsem1).wait()


def _embed(xf, tokc, pos_table):
    mesh = plsc.VectorSubcoreMesh(core_axis_name="c", subcore_axis_name="s")
    return pl.kernel(
        _embed_kernel,
        out_type=jax.ShapeDtypeStruct((_ROWS // 2, 2 * _EMBED), jnp.float32),
        mesh=mesh,
        scratch_types=[
            pltpu.VMEM((_RPW,), jnp.int32),
            pltpu.VMEM((_L, _EMBED), jnp.float32),
            pltpu.VMEM((_L, _EMBED), jnp.float32),
            pltpu.VMEM((_L // 2, 2 * _EMBED), jnp.float32),
            pltpu.VMEM((_L // 2, 2 * _EMBED), jnp.float32),
            pltpu.VMEM((_L, _EMBED), jnp.float32),
            pltpu.SemaphoreType.DMA,
            pltpu.SemaphoreType.DMA,
            pltpu.SemaphoreType.DMA,
            pltpu.SemaphoreType.DMA,
        ],
        compiler_params=pltpu.CompilerParams(use_tc_tiling_on_sc=False),
    )(xf, tokc, pos_table)


@jax.jit
def _run(x, tok_table, pos_table):
    xf = jnp.reshape(x, (_ROWS,)).astype(jnp.int32)
    tokw = _repack(jnp.transpose(tok_table))
    tokc = jnp.reshape(tokw, (_TROWS, _EMBED))
    out = _embed(xf, tokc, pos_table)
    return jnp.reshape(out, (_B, _L, _EMBED))


def kernel(x, tok_table, pos_table):
    return _run(x, tok_table, pos_table)
